# baseline (device time: 61133 ns/iter reference)
import jax
import jax.numpy as jnp
from jax import lax
from jax.experimental import pallas as pl
from jax.experimental.pallas import tpu as pltpu

N_DEV = 4
B = 2
SQ = 512
SKV = 512
H_LOC = 8
DH = 64
D_LOC = H_LOC * DH
D_MODEL = 768
BLK = 64
ROWS = B * SQ
STRIP = 128


def kernel(x, Wq, K_ext, V_ext, Wo):
    def body(x_ref, wq_ref, k_ref, v_ref, wo_ref, out_ref,
             p_ref, stage1, stage2, send_sems, recv_sems):
        my = lax.axis_index("i")
        xp = 3 - my
        yp = my ^ 1
        xc = my // 2
        yc = xc ^ (my % 2)

        barrier = pltpu.get_barrier_semaphore()
        for nbr in (xp, yp):
            pl.semaphore_signal(
                barrier, inc=1,
                device_id=(nbr,), device_id_type=pl.DeviceIdType.MESH,
            )
        pl.semaphore_wait(barrier, 2)

        qb = lax.broadcasted_iota(jnp.int32, (SQ, SKV), 0) // BLK
        kb = lax.broadcasted_iota(jnp.int32, (SQ, SKV), 1) // BLK
        mask = kb <= qb

        wq_loc = (wq_ref[:, pl.ds(my * D_LOC, D_LOC)] * 0.125).astype(
            jnp.bfloat16)
        wo_loc = wo_ref[pl.ds(my * D_LOC, D_LOC), :].astype(jnp.bfloat16)

        def attn_strip(b, r, q_b):
            r0 = r * STRIP
            kend = (r + 1) * STRIP
            heads = []
            for h in range(H_LOC):
                qh = q_b[r0:r0 + STRIP, h * DH:(h + 1) * DH].astype(
                    jnp.bfloat16)
                kh = k_ref[b, 0:kend, h, :].astype(jnp.bfloat16)
                vh = v_ref[b, 0:kend, h, :].astype(jnp.bfloat16)
                s = lax.dot_general(
                    qh, kh, (((1,), (1,)), ((), ())),
                    preferred_element_type=jnp.float32,
                )
                s = jnp.where(mask[r0:r0 + STRIP, 0:kend], s, -1e9)
                m = jnp.max(s, axis=-1, keepdims=True)
                w = jnp.exp(s - m)
                rs = 1.0 / jnp.sum(w, axis=-1, keepdims=True)
                ctx = jnp.dot(w.astype(jnp.bfloat16), vh,
                              preferred_element_type=jnp.float32)
                heads.append((ctx * rs).astype(jnp.bfloat16))
            return jnp.concatenate(heads, axis=1)

        def quarter_partial(b, half, q_b):
            ctx = jnp.concatenate(
                [attn_strip(b, 2 * half, q_b),
                 attn_strip(b, 2 * half + 1, q_b)], axis=0)
            return jnp.dot(ctx, wo_loc,
                           preferred_element_type=jnp.float32)

        def q_for_batch(b):
            return jnp.dot(x_ref[b].astype(jnp.bfloat16), wq_loc,
                           preferred_element_type=jnp.float32)

        def chain_geom(q):
            if q % 2 == 0:
                c, first, second = xc, xp, yp
            else:
                c, first, second = yc, yp, xp
            keep = q * 256 + c * STRIP
            send = q * 256 + (1 - c) * STRIP
            return keep, send, first, second

        def exch(idx, src_start, peer, dst_ref, dst_start=None):
            dst = dst_ref if dst_start is None else dst_ref.at[
                pl.ds(dst_start, STRIP)]
            return pltpu.make_async_remote_copy(
                src_ref=p_ref.at[pl.ds(src_start, STRIP)],
                dst_ref=dst,
                send_sem=send_sems.at[idx],
                recv_sem=recv_sems.at[idx],
                device_id=(peer,),
                device_id_type=pl.DeviceIdType.MESH,
            )

        def add_into(start, stage):
            blk = p_ref[pl.ds(start, STRIP), :].astype(jnp.float32)
            blk = blk + stage[...].astype(jnp.float32)
            p_ref[pl.ds(start, STRIP), :] = blk.astype(jnp.bfloat16)

        def start1(q):
            keep, send, first, _ = chain_geom(q)
            r = exch(3 * q + 0, send, first, stage1.at[q])
            r.start()
            return r

        def svc1(q, r):
            keep, _, _, second = chain_geom(q)
            r.wait()
            add_into(keep, stage1.at[q])
            r2 = exch(3 * q + 1, keep, second, stage2.at[q])
            r2.start()
            return r2

        def svc2(q, r2):
            keep, _, first, _ = chain_geom(q)
            r2.wait()
            add_into(keep, stage2.at[q])
            r3 = exch(3 * q + 2, keep, first, p_ref, keep)
            r3.start()
            return r3

        q_b0 = q_for_batch(0)
        p_ref[pl.ds(0, 256), :] = quarter_partial(0, 0, q_b0).astype(
            jnp.bfloat16)
        c0 = start1(0)

        p_ref[pl.ds(256, 256), :] = quarter_partial(0, 1, q_b0).astype(
            jnp.bfloat16)
        c1 = start1(1)
        c0 = svc1(0, c0)

        q_b1 = q_for_batch(1)
        p_ref[pl.ds(512, 256), :] = quarter_partial(1, 0, q_b1).astype(
            jnp.bfloat16)
        c2 = start1(2)
        c1 = svc1(1, c1)
        c0 = svc2(0, c0)

        p_ref[pl.ds(768, 256), :] = quarter_partial(1, 1, q_b1).astype(
            jnp.bfloat16)
        c3 = start1(3)
        c2 = svc1(2, c2)
        c1 = svc2(1, c1)

        c3 = svc1(3, c3)
        c2 = svc2(2, c2)
        c3 = svc2(3, c3)

        c0.wait()
        c1.wait()
        c2.wait()
        c3.wait()

        out_ref[0] = p_ref[pl.ds(0, SQ), :].astype(jnp.float32)
        out_ref[1] = p_ref[pl.ds(SQ, SQ), :].astype(jnp.float32)

    return pl.pallas_call(
        body,
        out_shape=jax.ShapeDtypeStruct((B, SQ, D_MODEL), jnp.float32),
        in_specs=[pl.BlockSpec(memory_space=pltpu.VMEM)] * 5,
        out_specs=pl.BlockSpec(memory_space=pltpu.VMEM),
        scratch_shapes=[
            pltpu.VMEM((ROWS, D_MODEL), jnp.bfloat16),
            pltpu.VMEM((4, STRIP, D_MODEL), jnp.bfloat16),
            pltpu.VMEM((4, STRIP, D_MODEL), jnp.bfloat16),
            pltpu.SemaphoreType.DMA((12,)),
            pltpu.SemaphoreType.DMA((12,)),
        ],
        compiler_params=pltpu.CompilerParams(collective_id=0),
    )(x, Wq, K_ext, V_ext, Wo)


# device time: 43862 ns/iter; 1.3938x vs baseline; 1.3938x over previous
import jax
import jax.numpy as jnp
from jax import lax
from jax.experimental import pallas as pl
from jax.experimental.pallas import tpu as pltpu

N_DEV = 4
B = 2
SQ = 512
SKV = 512
H_LOC = 8
DH = 64
D_LOC = H_LOC * DH
D_MODEL = 768
BLK = 64
ROWS = B * SQ
PIECE = 64


def kernel(x, Wq, K_ext, V_ext, Wo):
    def body(x_ref, wq_ref, k_ref, v_ref, wo_ref, out_ref,
             p_ref, scat, send_sems, recv_sems):
        my = lax.axis_index("i")
        xp = 3 - my
        yp = my ^ 1
        dg = my ^ 2
        xc = my // 2
        yc = xc ^ (my % 2)
        o_me = 2 * xc + yc

        barrier = pltpu.get_barrier_semaphore()
        for nbr in (xp, yp):
            pl.semaphore_signal(
                barrier, inc=1,
                device_id=(nbr,), device_id_type=pl.DeviceIdType.MESH,
            )
        pl.semaphore_wait(barrier, 2)

        qb = lax.broadcasted_iota(jnp.int32, (SQ, SKV), 0) // BLK
        kb = lax.broadcasted_iota(jnp.int32, (SQ, SKV), 1) // BLK
        mask = kb <= qb

        wq_loc = (wq_ref[:, pl.ds(my * D_LOC, D_LOC)] * 0.125).astype(
            jnp.bfloat16)
        wo_loc = wo_ref[pl.ds(my * D_LOC, D_LOC), :].astype(jnp.bfloat16)

        def partial_for_batch(b):
            xb = x_ref[b].astype(jnp.bfloat16)
            q_b = jnp.dot(xb, wq_loc,
                          preferred_element_type=jnp.float32)
            heads = []
            for h in range(H_LOC):
                qh = q_b[:, h * DH:(h + 1) * DH].astype(jnp.bfloat16)
                kh = k_ref[b, :, h, :].astype(jnp.bfloat16)
                vh = v_ref[b, :, h, :].astype(jnp.bfloat16)
                s = lax.dot_general(
                    qh, kh, (((1,), (1,)), ((), ())),
                    preferred_element_type=jnp.float32,
                )
                w = jnp.exp(jnp.where(mask, s, -1e9))
                rs = 1.0 / jnp.sum(w, axis=-1, keepdims=True)
                ctx = jnp.dot(w.astype(jnp.bfloat16), vh,
                              preferred_element_type=jnp.float32)
                heads.append((ctx * rs).astype(jnp.bfloat16))
            ctx_b = jnp.concatenate(heads, axis=1)
            return jnp.dot(ctx_b, wo_loc,
                           preferred_element_type=jnp.float32)

        def peers():
            return ((xp, o_me ^ 2, 0), (yp, o_me ^ 1, 1), (dg, o_me ^ 3, 2))

        def rdma(idx, src_start, peer, dst_ref, dst_start=None):
            dst = dst_ref if dst_start is None else dst_ref.at[
                pl.ds(dst_start, PIECE)]
            return pltpu.make_async_remote_copy(
                src_ref=p_ref.at[pl.ds(src_start, PIECE)],
                dst_ref=dst,
                send_sem=send_sems.at[idx],
                recv_sem=recv_sems.at[idx],
                device_id=(peer,),
                device_id_type=pl.DeviceIdType.MESH,
            )

        def scatter(q):
            q0 = q * 256
            descs = []
            for peer, o, r in peers():
                d = rdma(6 * q + r, q0 + o * PIECE, peer, scat.at[q, r])
                d.start()
                descs.append(d)
            return descs

        def reduce_and_gather(q, descs):
            for d in descs:
                d.wait()
            mine = q * 256 + o_me * PIECE
            acc = p_ref[pl.ds(mine, PIECE), :].astype(jnp.float32)
            for r in range(3):
                acc = acc + scat[q, r].astype(jnp.float32)
            p_ref[pl.ds(mine, PIECE), :] = acc.astype(jnp.bfloat16)
            gds = []
            for peer, _, r in peers():
                d = rdma(6 * q + 3 + r, mine, peer, p_ref, mine)
                d.start()
                gds.append(d)
            return gds

        p_ref[pl.ds(0, SQ), :] = partial_for_batch(0).astype(jnp.bfloat16)
        s0 = scatter(0)
        s1 = scatter(1)

        p_ref[pl.ds(SQ, SQ), :] = partial_for_batch(1).astype(jnp.bfloat16)
        s2 = scatter(2)
        s3 = scatter(3)

        g0 = reduce_and_gather(0, s0)
        g1 = reduce_and_gather(1, s1)
        g2 = reduce_and_gather(2, s2)
        g3 = reduce_and_gather(3, s3)

        for gds in (g0, g1, g2, g3):
            for d in gds:
                d.wait()

        out_ref[0] = p_ref[pl.ds(0, SQ), :].astype(jnp.float32)
        out_ref[1] = p_ref[pl.ds(SQ, SQ), :].astype(jnp.float32)

    return pl.pallas_call(
        body,
        out_shape=jax.ShapeDtypeStruct((B, SQ, D_MODEL), jnp.float32),
        in_specs=[pl.BlockSpec(memory_space=pltpu.VMEM)] * 5,
        out_specs=pl.BlockSpec(memory_space=pltpu.VMEM),
        scratch_shapes=[
            pltpu.VMEM((ROWS, D_MODEL), jnp.bfloat16),
            pltpu.VMEM((4, 3, PIECE, D_MODEL), jnp.bfloat16),
            pltpu.SemaphoreType.DMA((24,)),
            pltpu.SemaphoreType.DMA((24,)),
        ],
        compiler_params=pltpu.CompilerParams(collective_id=0),
    )(x, Wq, K_ext, V_ext, Wo)


# device time: 34049 ns/iter; 1.7954x vs baseline; 1.2882x over previous
import jax
import jax.numpy as jnp
from jax import lax
from jax.experimental import pallas as pl
from jax.experimental.pallas import tpu as pltpu

N_DEV = 4
B = 2
SQ = 512
SKV = 512
H_LOC = 8
DH = 64
D_LOC = H_LOC * DH
D_MODEL = 768
BLK = 64
ROWS = B * SQ
PIECE = 64


def kernel(x, Wq, K_ext, V_ext, Wo):
    my_out = lax.axis_index("i")
    xb = x.astype(jnp.bfloat16)
    wq_loc = (lax.dynamic_slice_in_dim(Wq, my_out * D_LOC, D_LOC, axis=1)
              * 0.125).astype(jnp.bfloat16)
    wo_loc = lax.dynamic_slice_in_dim(Wo, my_out * D_LOC, D_LOC, axis=0
                                      ).astype(jnp.bfloat16)
    kb = jnp.transpose(K_ext.astype(jnp.bfloat16), (0, 2, 1, 3))
    vb = jnp.transpose(V_ext.astype(jnp.bfloat16), (0, 2, 1, 3))

    def body(x_ref, wq_ref, k_ref, v_ref, wo_ref, out_ref,
             p_ref, scat, send_sems, recv_sems):
        my = lax.axis_index("i")
        xp = 3 - my
        yp = my ^ 1
        dg = my ^ 2
        xc = my // 2
        yc = xc ^ (my % 2)
        o_me = 2 * xc + yc

        barrier = pltpu.get_barrier_semaphore()
        for nbr in (xp, yp):
            pl.semaphore_signal(
                barrier, inc=1,
                device_id=(nbr,), device_id_type=pl.DeviceIdType.MESH,
            )
        pl.semaphore_wait(barrier, 2)

        qb_i = lax.broadcasted_iota(jnp.int32, (SQ, SKV), 0) // BLK
        kb_i = lax.broadcasted_iota(jnp.int32, (SQ, SKV), 1) // BLK
        mask = kb_i <= qb_i

        def q_for_batch(b):
            return jnp.dot(x_ref[b], wq_ref[...],
                           preferred_element_type=jnp.float32
                           ).astype(jnp.bfloat16)

        def ctx_heads(b, q_b, hs):
            heads = []
            for h in hs:
                qh = q_b[:, h * DH:(h + 1) * DH]
                s = lax.dot_general(
                    qh, k_ref[b, h], (((1,), (1,)), ((), ())),
                    preferred_element_type=jnp.float32,
                )
                w = jnp.exp(jnp.where(mask, s, -1e9).astype(jnp.bfloat16))
                rs = 1.0 / jnp.sum(w, axis=-1, keepdims=True,
                                   dtype=jnp.float32)
                ctx = jnp.dot(w, v_ref[b, h],
                              preferred_element_type=jnp.float32)
                heads.append((ctx * rs).astype(jnp.bfloat16))
            return heads

        def partial_slab(ctx_b, lo):
            return jnp.dot(ctx_b[lo:lo + 256, :], wo_ref[...],
                           preferred_element_type=jnp.float32
                           ).astype(jnp.bfloat16)

        def peers():
            return ((xp, o_me ^ 2, 0), (yp, o_me ^ 1, 1), (dg, o_me ^ 3, 2))

        def rdma(idx, src_start, peer, dst_ref, dst_start=None):
            dst = dst_ref if dst_start is None else dst_ref.at[
                pl.ds(dst_start, PIECE)]
            return pltpu.make_async_remote_copy(
                src_ref=p_ref.at[pl.ds(src_start, PIECE)],
                dst_ref=dst,
                send_sem=send_sems.at[idx],
                recv_sem=recv_sems.at[idx],
                device_id=(peer,),
                device_id_type=pl.DeviceIdType.MESH,
            )

        def scatter(q):
            q0 = q * 256
            descs = []
            for peer, o, r in peers():
                d = rdma(6 * q + r, q0 + o * PIECE, peer, scat.at[q, r])
                d.start()
                descs.append(d)
            return descs

        def reduce_and_gather(q, descs):
            for d in descs:
                d.wait()
            mine = q * 256 + o_me * PIECE
            acc = p_ref[pl.ds(mine, PIECE), :].astype(jnp.float32)
            for r in range(3):
                acc = acc + scat[q, r].astype(jnp.float32)
            p_ref[pl.ds(mine, PIECE), :] = acc.astype(jnp.bfloat16)
            gds = []
            for peer, _, r in peers():
                d = rdma(6 * q + 3 + r, mine, peer, p_ref, mine)
                d.start()
                gds.append(d)
            return gds

        q_b0 = q_for_batch(0)
        ctx_b0 = jnp.concatenate(ctx_heads(0, q_b0, range(H_LOC)), axis=1)
        p_ref[pl.ds(0, 256), :] = partial_slab(ctx_b0, 0)
        s0 = scatter(0)
        p_ref[pl.ds(256, 256), :] = partial_slab(ctx_b0, 256)
        s1 = scatter(1)

        q_b1 = q_for_batch(1)
        h1a = ctx_heads(1, q_b1, range(0, 4))
        g0 = reduce_and_gather(0, s0)
        h1b = ctx_heads(1, q_b1, range(4, H_LOC))
        g1 = reduce_and_gather(1, s1)
        ctx_b1 = jnp.concatenate(h1a + h1b, axis=1)
        p_ref[pl.ds(512, 256), :] = partial_slab(ctx_b1, 0)
        s2 = scatter(2)
        p_ref[pl.ds(768, 256), :] = partial_slab(ctx_b1, 256)
        s3 = scatter(3)

        g2 = reduce_and_gather(2, s2)
        g3 = reduce_and_gather(3, s3)

        for d in g0 + g1:
            d.wait()
        out_ref[0] = p_ref[pl.ds(0, SQ), :].astype(jnp.float32)
        for d in g2 + g3:
            d.wait()
        out_ref[1] = p_ref[pl.ds(SQ, SQ), :].astype(jnp.float32)

    return pl.pallas_call(
        body,
        out_shape=jax.ShapeDtypeStruct((B, SQ, D_MODEL), jnp.float32),
        in_specs=[pl.BlockSpec(memory_space=pltpu.VMEM)] * 5,
        out_specs=pl.BlockSpec(memory_space=pltpu.VMEM),
        scratch_shapes=[
            pltpu.VMEM((ROWS, D_MODEL), jnp.bfloat16),
            pltpu.VMEM((4, 3, PIECE, D_MODEL), jnp.bfloat16),
            pltpu.SemaphoreType.DMA((24,)),
            pltpu.SemaphoreType.DMA((24,)),
        ],
        compiler_params=pltpu.CompilerParams(collective_id=0),
    )(xb, wq_loc, kb, vb, wo_loc)


# device time: 32933 ns/iter; 1.8563x vs baseline; 1.0339x over previous
import jax
import jax.numpy as jnp
from jax import lax
from jax.experimental import pallas as pl
from jax.experimental.pallas import tpu as pltpu

N_DEV = 4
B = 2
SQ = 512
SKV = 512
H_LOC = 8
DH = 64
D_LOC = H_LOC * DH
D_MODEL = 768
BLK = 64
ROWS = B * SQ
PIECE = 64


def kernel(x, Wq, K_ext, V_ext, Wo):
    my_out = lax.axis_index("i")
    xb = x.astype(jnp.bfloat16)
    wq_loc = (lax.dynamic_slice_in_dim(Wq, my_out * D_LOC, D_LOC, axis=1)
              * 0.125).astype(jnp.bfloat16)
    wo_loc = lax.dynamic_slice_in_dim(Wo, my_out * D_LOC, D_LOC, axis=0
                                      ).astype(jnp.bfloat16)
    kb = jnp.transpose(K_ext.astype(jnp.bfloat16), (0, 2, 1, 3))
    vb = jnp.transpose(V_ext.astype(jnp.bfloat16), (0, 2, 1, 3))

    def body(x_ref, wq_ref, k_ref, v_ref, wo_ref, out_ref,
             p_ref, scat, send_sems, recv_sems):
        my = lax.axis_index("i")
        xp = 3 - my
        yp = my ^ 1
        dg = my ^ 2
        xc = my // 2
        yc = xc ^ (my % 2)
        o_me = 2 * xc + yc

        barrier = pltpu.get_barrier_semaphore()
        for nbr in (xp, yp):
            pl.semaphore_signal(
                barrier, inc=1,
                device_id=(nbr,), device_id_type=pl.DeviceIdType.MESH,
            )
        pl.semaphore_wait(barrier, 2)

        qb_i = lax.broadcasted_iota(jnp.int32, (SQ, SKV), 0) // BLK
        kb_i = lax.broadcasted_iota(jnp.int32, (SQ, SKV), 1) // BLK
        mask = kb_i <= qb_i

        def q_for_batch(b):
            return jnp.dot(x_ref[b], wq_ref[...],
                           preferred_element_type=jnp.float32
                           ).astype(jnp.bfloat16)

        def ctx_rows(b, q_b, r0, kend):
            heads = []
            for h in range(H_LOC):
                qh = q_b[r0:r0 + 256, h * DH:(h + 1) * DH]
                s = lax.dot_general(
                    qh, k_ref[b, h, 0:kend, :], (((1,), (1,)), ((), ())),
                    preferred_element_type=jnp.float32,
                )
                w = jnp.exp(jnp.where(mask[r0:r0 + 256, 0:kend], s,
                                      -1e9).astype(jnp.bfloat16))
                rs = 1.0 / jnp.sum(w, axis=-1, keepdims=True,
                                   dtype=jnp.float32)
                ctx = jnp.dot(w, v_ref[b, h, 0:kend, :],
                              preferred_element_type=jnp.float32)
                heads.append((ctx * rs).astype(jnp.bfloat16))
            return jnp.concatenate(heads, axis=1)

        def partial_slab(ctx256):
            return jnp.dot(ctx256, wo_ref[...],
                           preferred_element_type=jnp.float32
                           ).astype(jnp.bfloat16)

        def peers():
            return ((xp, o_me ^ 2, 0), (yp, o_me ^ 1, 1), (dg, o_me ^ 3, 2))

        def rdma(idx, src_start, peer, dst_ref, dst_start=None):
            dst = dst_ref if dst_start is None else dst_ref.at[
                pl.ds(dst_start, PIECE)]
            return pltpu.make_async_remote_copy(
                src_ref=p_ref.at[pl.ds(src_start, PIECE)],
                dst_ref=dst,
                send_sem=send_sems.at[idx],
                recv_sem=recv_sems.at[idx],
                device_id=(peer,),
                device_id_type=pl.DeviceIdType.MESH,
            )

        def scatter(q):
            q0 = q * 256
            descs = []
            for peer, o, r in peers():
                d = rdma(6 * q + r, q0 + o * PIECE, peer, scat.at[q, r])
                d.start()
                descs.append(d)
            return descs

        def reduce_and_gather(q, descs):
            for d in descs:
                d.wait()
            mine = q * 256 + o_me * PIECE
            acc = p_ref[pl.ds(mine, PIECE), :].astype(jnp.float32)
            for r in range(3):
                acc = acc + scat[q, r].astype(jnp.float32)
            p_ref[pl.ds(mine, PIECE), :] = acc.astype(jnp.bfloat16)
            gds = []
            for peer, _, r in peers():
                d = rdma(6 * q + 3 + r, mine, peer, p_ref, mine)
                d.start()
                gds.append(d)
            return gds

        q_b0 = q_for_batch(0)
        p_ref[pl.ds(0, 256), :] = partial_slab(ctx_rows(0, q_b0, 0, 256))
        s0 = scatter(0)
        p_ref[pl.ds(256, 256), :] = partial_slab(ctx_rows(0, q_b0, 256, SKV))
        s1 = scatter(1)

        q_b1 = q_for_batch(1)
        p_ref[pl.ds(512, 256), :] = partial_slab(ctx_rows(1, q_b1, 0, 256))
        s2 = scatter(2)
        g0 = reduce_and_gather(0, s0)
        p_ref[pl.ds(768, 256), :] = partial_slab(ctx_rows(1, q_b1, 256, SKV))
        s3 = scatter(3)
        g1 = reduce_and_gather(1, s1)

        g2 = reduce_and_gather(2, s2)
        g3 = reduce_and_gather(3, s3)

        for d in g0 + g1:
            d.wait()
        out_ref[0] = p_ref[pl.ds(0, SQ), :].astype(jnp.float32)
        for d in g2 + g3:
            d.wait()
        out_ref[1] = p_ref[pl.ds(SQ, SQ), :].astype(jnp.float32)

    return pl.pallas_call(
        body,
        out_shape=jax.ShapeDtypeStruct((B, SQ, D_MODEL), jnp.float32),
        in_specs=[pl.BlockSpec(memory_space=pltpu.VMEM)] * 5,
        out_specs=pl.BlockSpec(memory_space=pltpu.VMEM),
        scratch_shapes=[
            pltpu.VMEM((ROWS, D_MODEL), jnp.bfloat16),
            pltpu.VMEM((4, 3, PIECE, D_MODEL), jnp.bfloat16),
            pltpu.SemaphoreType.DMA((24,)),
            pltpu.SemaphoreType.DMA((24,)),
        ],
        compiler_params=pltpu.CompilerParams(collective_id=0),
    )(xb, wq_loc, kb, vb, wo_loc)
